# baseline (speedup 1.0000x reference)
import functools

import jax
import jax.numpy as jnp
import numpy as np
from jax.experimental import pallas as pl
from jax.experimental.pallas import tpu as pltpu

_P = 0.1
_VOCAB = 1000
_SEED = 42

_CH = 8
_NBUF = 4


@functools.lru_cache(maxsize=None)
def _draws(B, L):
    cpu = jax.devices("cpu")[0]
    with jax.ensure_compile_time_eval(), jax.default_device(cpu):
        key = jax.random.key(_SEED)
        k1, k2 = jax.random.split(key)
        tgt = jax.random.uniform(k1, (B, L, 1)) < _P
        rep = jax.random.randint(k2, (B, L, 1), 0, _VOCAB - 2)
    return (np.asarray(tgt, dtype=np.int32),
            np.asarray(rep, dtype=np.int32))


def _process(m, tgt, rep, an):
    mx = jnp.max(m, axis=2, keepdims=True)
    lane = jax.lax.broadcasted_iota(jnp.int32, m.shape, 2)
    idx = jnp.min(jnp.where(m == mx, lane, jnp.int32(2**30)),
                  axis=2, keepdims=True)
    msg_exp = jnp.maximum(idx, 1)
    repl_sym = jnp.where(rep + 1 < msg_exp, rep + 1, rep + 2)
    combined = (tgt != 0) & (idx != 0) & (an != 0)
    onehot = (lane == repl_sym).astype(m.dtype)
    return jnp.where(combined, onehot, m)


def _channel_kernel(an_ref, msg_hbm, tgt_ref, rep_ref, out_hbm,
                    buf_in, buf_out, sem_in, sem_out):
    B = msg_hbm.shape[0]
    nch = B // _CH
    an = an_ref[0]

    def in_copy(i, slot):
        return pltpu.make_async_copy(
            msg_hbm.at[pl.ds(i * _CH, _CH)], buf_in.at[slot], sem_in.at[slot])

    def out_copy(i, slot):
        return pltpu.make_async_copy(
            buf_out.at[slot], out_hbm.at[pl.ds(i * _CH, _CH)], sem_out.at[slot])

    for k in range(min(_NBUF, nch)):
        in_copy(k, k).start()

    for i in range(nch):
        slot = i % _NBUF
        in_copy(i, slot).wait()
        if i >= _NBUF:
            out_copy(i - _NBUF, slot).wait()
        sl = pl.ds(i * _CH, _CH)
        buf_out[slot] = _process(buf_in[slot], tgt_ref[sl], rep_ref[sl], an)
        out_copy(i, slot).start()
        nxt = i + _NBUF
        if nxt < nch:
            in_copy(nxt, slot).start()

    for i in range(max(0, nch - _NBUF), nch):
        out_copy(i, i % _NBUF).wait()


@jax.jit
def kernel(message, apply_noise):
    B, L, V = message.shape
    tgt, rep = _draws(B, L)
    an = jnp.asarray(apply_noise, jnp.int32).reshape(1)

    return pl.pallas_call(
        _channel_kernel,
        in_specs=[
            pl.BlockSpec(memory_space=pltpu.MemorySpace.SMEM),
            pl.BlockSpec(memory_space=pltpu.MemorySpace.HBM),
            pl.BlockSpec(memory_space=pltpu.MemorySpace.VMEM),
            pl.BlockSpec(memory_space=pltpu.MemorySpace.VMEM),
        ],
        out_specs=pl.BlockSpec(memory_space=pltpu.MemorySpace.HBM),
        out_shape=jax.ShapeDtypeStruct((B, L, V), message.dtype),
        scratch_shapes=[
            pltpu.VMEM((_NBUF, _CH, L, V), message.dtype),
            pltpu.VMEM((_NBUF, _CH, L, V), message.dtype),
            pltpu.SemaphoreType.DMA((_NBUF,)),
            pltpu.SemaphoreType.DMA((_NBUF,)),
        ],
    )(an, message, jnp.asarray(tgt), jnp.asarray(rep))


# speedup vs baseline: 1.0233x; 1.0233x over previous
import functools

import jax
import jax.numpy as jnp
import numpy as np
from jax.experimental import pallas as pl
from jax.experimental.pallas import tpu as pltpu

_P = 0.1
_VOCAB = 1000
_SEED = 42

_CH = 16
_NBUF = 4


@functools.lru_cache(maxsize=None)
def _draws(B, L):
    cpu = jax.devices("cpu")[0]
    with jax.ensure_compile_time_eval(), jax.default_device(cpu):
        key = jax.random.key(_SEED)
        k1, k2 = jax.random.split(key)
        tgt = jax.random.uniform(k1, (B, L, 1)) < _P
        rep = jax.random.randint(k2, (B, L, 1), 0, _VOCAB - 2)
    return (np.asarray(tgt, dtype=np.int32),
            np.asarray(rep, dtype=np.int32))


def _process(m, tgt, rep, an):
    mx = jnp.max(m, axis=2, keepdims=True)
    lane = jax.lax.broadcasted_iota(jnp.int32, m.shape, 2)
    idx = jnp.min(jnp.where(m == mx, lane, jnp.int32(2**30)),
                  axis=2, keepdims=True)
    msg_exp = jnp.maximum(idx, 1)
    repl_sym = jnp.where(rep + 1 < msg_exp, rep + 1, rep + 2)
    combined = (tgt != 0) & (idx != 0) & (an != 0)
    onehot = (lane == repl_sym).astype(m.dtype)
    return jnp.where(combined, onehot, m)


def _channel_kernel(an_ref, msg_hbm, tgt_ref, rep_ref, out_hbm,
                    buf_in, buf_out, sem_in, sem_out):
    B = msg_hbm.shape[0]
    nch = B // _CH
    an = an_ref[0]

    def in_copy(i, slot):
        return pltpu.make_async_copy(
            msg_hbm.at[pl.ds(i * _CH, _CH)], buf_in.at[slot], sem_in.at[slot])

    def out_copy(i, slot):
        return pltpu.make_async_copy(
            buf_out.at[slot], out_hbm.at[pl.ds(i * _CH, _CH)], sem_out.at[slot])

    for k in range(min(_NBUF, nch)):
        in_copy(k, k).start()

    for i in range(nch):
        slot = i % _NBUF
        in_copy(i, slot).wait()
        if i >= _NBUF:
            out_copy(i - _NBUF, slot).wait()
        sl = pl.ds(i * _CH, _CH)
        buf_out[slot] = _process(buf_in[slot], tgt_ref[sl], rep_ref[sl], an)
        out_copy(i, slot).start()
        nxt = i + _NBUF
        if nxt < nch:
            in_copy(nxt, slot).start()

    for i in range(max(0, nch - _NBUF), nch):
        out_copy(i, i % _NBUF).wait()


@jax.jit
def kernel(message, apply_noise):
    B, L, V = message.shape
    tgt, rep = _draws(B, L)
    an = jnp.asarray(apply_noise, jnp.int32).reshape(1)

    return pl.pallas_call(
        _channel_kernel,
        in_specs=[
            pl.BlockSpec(memory_space=pltpu.MemorySpace.SMEM),
            pl.BlockSpec(memory_space=pltpu.MemorySpace.HBM),
            pl.BlockSpec(memory_space=pltpu.MemorySpace.VMEM),
            pl.BlockSpec(memory_space=pltpu.MemorySpace.VMEM),
        ],
        out_specs=pl.BlockSpec(memory_space=pltpu.MemorySpace.HBM),
        out_shape=jax.ShapeDtypeStruct((B, L, V), message.dtype),
        scratch_shapes=[
            pltpu.VMEM((_NBUF, _CH, L, V), message.dtype),
            pltpu.VMEM((_NBUF, _CH, L, V), message.dtype),
            pltpu.SemaphoreType.DMA((_NBUF,)),
            pltpu.SemaphoreType.DMA((_NBUF,)),
        ],
    )(an, message, jnp.asarray(tgt), jnp.asarray(rep))
